# Initial kernel scaffold; baseline (speedup 1.0000x reference)
import jax
import jax.numpy as jnp
from jax.experimental import pallas as pl


def kernel(queries, keys, values, attn_mask):
    raise NotImplementedError("write your pallas kernel here")


# speedup vs baseline: 2.7711x; 2.7711x over previous
import math
import jax
import jax.numpy as jnp
from jax.experimental import pallas as pl
from jax.experimental.pallas import tpu as pltpu


def _corr_body(nb, t, q_ref, k_ref, low_ref, up_ref, acc_ref):
    a = pl.program_id(2)
    x = q_ref[0, 0]
    y = k_ref[0, 0]
    g = jax.lax.dot_general(
        x, y, (((1,), (1,)), ((), ())),
        precision=jax.lax.Precision.HIGHEST,
        preferred_element_type=jnp.float32,
    )

    @pl.when(a == 0)
    def _():
        acc_ref[...] = g

    @pl.when(a > 0)
    def _():
        acc_ref[...] = acc_ref[...] + g

    @pl.when(a == nb - 1)
    def _():
        rot = acc_ref[...]
        rr = jax.lax.broadcasted_iota(jnp.int32, (t, t), 0)
        cc = jax.lax.broadcasted_iota(jnp.int32, (t, t), 1)
        for bit in range(int(math.log2(t))):
            amt = 1 << bit
            rot = jnp.where((cc & amt) != 0, jnp.roll(rot, -amt, axis=0), rot)
        nowrap = (rr + cc) < t
        low_ref[0, 0] = jnp.sum(jnp.where(nowrap, rot, 0.0), axis=1)
        up_ref[0, 0] = jnp.sum(jnp.where(nowrap, 0.0, rot), axis=1)


def _select_body(top_k, t, scale, low_ref, up_ref, idx_ref, w_ref):
    bsz, ll = low_ref.shape
    mv = (low_ref[...] + jnp.roll(up_ref[...], -t, axis=1)) * scale
    bm = jnp.mean(mv, axis=0, keepdims=True)
    lane = jax.lax.broadcasted_iota(jnp.int32, (1, ll), 1)
    idx_parts = []
    w_parts = []
    for _ in range(top_k):
        m = jnp.max(bm)
        idx = jnp.min(jnp.where(bm == m, lane, ll))
        idx_parts.append(jnp.full((1, 1), idx, jnp.int32))
        w_parts.append(
            jnp.sum(jnp.where(lane == idx, mv, 0.0), axis=1, keepdims=True))
        bm = jnp.where(lane == idx, -3.0e38, bm)
    w = jnp.concatenate(w_parts, axis=1)
    w = jax.nn.softmax(w, axis=-1)
    idx_ref[...] = jnp.concatenate(idx_parts, axis=1)
    w_ref[...] = w


def _agg_body(ll, top_k, tl, idx_ref, w_ref, v_ref, out_ref):
    b = pl.program_id(0)
    vcur = v_ref[0]
    shift0 = jax.lax.rem(ll - idx_ref[0, 0], ll)
    out_ref[0] = w_ref[b, 0] * pltpu.roll(vcur, shift0, axis=0)
    for i in range(1, top_k):
        shift = jax.lax.rem(ll - idx_ref[0, i], ll)
        out_ref[0] = out_ref[0] + w_ref[b, i] * pltpu.roll(vcur, shift, axis=0)


def kernel(queries, keys, values, attn_mask):
    B, L, H, E = queries.shape
    C = H * E
    T = 1024
    NB = L // T
    top_k = int(1 * math.log(L))

    q = queries.reshape(B, L, C)
    k = keys.reshape(B, L, C)
    v = values.reshape(B, L, C)

    import functools
    low, up = pl.pallas_call(
        functools.partial(_corr_body, NB, T),
        grid=(B, NB, NB),
        in_specs=[
            pl.BlockSpec((1, 1, T, C), lambda b, j, a: (b, (a + j) % NB, 0, 0)),
            pl.BlockSpec((1, 1, T, C), lambda b, j, a: (b, a, 0, 0)),
        ],
        out_specs=[
            pl.BlockSpec((1, 1, T), lambda b, j, a: (b * NB + j, 0, 0)),
            pl.BlockSpec((1, 1, T), lambda b, j, a: (b * NB + j, 0, 0)),
        ],
        out_shape=[
            jax.ShapeDtypeStruct((B * NB, 1, T), jnp.float32),
            jax.ShapeDtypeStruct((B * NB, 1, T), jnp.float32),
        ],
        scratch_shapes=[pltpu.VMEM((T, T), jnp.float32)],
        compiler_params=pltpu.CompilerParams(
            dimension_semantics=("parallel", "parallel", "arbitrary"),
        ),
    )(q.reshape(B, NB, T, C), k.reshape(B, NB, T, C))

    low = low.reshape(B, L)
    up = up.reshape(B, L)

    idx, w = pl.pallas_call(
        functools.partial(_select_body, top_k, T, 1.0 / C),
        in_specs=[
            pl.BlockSpec((B, L), lambda: (0, 0)),
            pl.BlockSpec((B, L), lambda: (0, 0)),
        ],
        out_specs=[
            pl.BlockSpec((1, top_k), lambda: (0, 0)),
            pl.BlockSpec((B, top_k), lambda: (0, 0)),
        ],
        out_shape=[
            jax.ShapeDtypeStruct((1, top_k), jnp.int32),
            jax.ShapeDtypeStruct((B, top_k), jnp.float32),
        ],
    )(low, up)

    CB = 128
    NCB = C // CB
    TL = 512
    out = pl.pallas_call(
        functools.partial(_agg_body, L, top_k, TL),
        grid=(B, NCB),
        in_specs=[
            pl.BlockSpec(memory_space=pltpu.SMEM),
            pl.BlockSpec(memory_space=pltpu.SMEM),
            pl.BlockSpec((1, L, CB), lambda b, c: (b, 0, c)),
        ],
        out_specs=pl.BlockSpec((1, L, CB), lambda b, c: (b, 0, c)),
        out_shape=jax.ShapeDtypeStruct((B, L, C), jnp.float32),
        compiler_params=pltpu.CompilerParams(
            dimension_semantics=("parallel", "parallel"),
        ),
    )(idx, w, v)

    return out.reshape(B, L, H, E)


# speedup vs baseline: 3.5655x; 1.2866x over previous
import functools
import math
import jax
import jax.numpy as jnp
from jax import lax
from jax.experimental import pallas as pl
from jax.experimental.pallas import tpu as pltpu
from jax.experimental.pallas import tpu_sc as plsc


def _corr_body(nb, t, q_ref, k_ref, low_ref, up_ref, acc_ref):
    a = pl.program_id(2)
    x = q_ref[0, 0]
    y = k_ref[0, 0]
    xh = x.astype(jnp.bfloat16)
    xl = (x - xh.astype(jnp.float32)).astype(jnp.bfloat16)
    yh = y.astype(jnp.bfloat16)
    yl = (y - yh.astype(jnp.float32)).astype(jnp.bfloat16)
    dims = (((1,), (1,)), ((), ()))

    def _mm(u, v):
        return jax.lax.dot_general(
            u, v, dims, preferred_element_type=jnp.float32)

    g = _mm(xh, yh) + _mm(xh, yl) + _mm(xl, yh)

    @pl.when(a == 0)
    def _():
        acc_ref[...] = g

    @pl.when(a > 0)
    def _():
        acc_ref[...] = acc_ref[...] + g

    @pl.when(a == nb - 1)
    def _():
        rot = acc_ref[...]
        rr = jax.lax.broadcasted_iota(jnp.int32, (t, t), 0)
        cc = jax.lax.broadcasted_iota(jnp.int32, (t, t), 1)
        for bit in range(int(math.log2(t))):
            amt = 1 << bit
            rot = jnp.where((cc & amt) != 0, jnp.roll(rot, -amt, axis=0), rot)
        nowrap = (rr + cc) < t
        low_ref[0, 0] = jnp.sum(jnp.where(nowrap, rot, 0.0), axis=1)
        up_ref[0, 0] = jnp.sum(jnp.where(nowrap, 0.0, rot), axis=1)


def _select_body(top_k, t, scale, low_ref, up_ref, idx_ref, w_ref):
    bsz, ll = low_ref.shape
    mv = (low_ref[...] + jnp.roll(up_ref[...], -t, axis=1)) * scale
    bm = jnp.mean(mv, axis=0, keepdims=True)
    lane = jax.lax.broadcasted_iota(jnp.int32, (1, ll), 1)
    idx_parts = []
    w_parts = []
    for _ in range(top_k):
        m = jnp.max(bm)
        idx = jnp.min(jnp.where(bm == m, lane, ll))
        idx_parts.append(jnp.full((1, 1), idx, jnp.int32))
        w_parts.append(
            jnp.sum(jnp.where(lane == idx, mv, 0.0), axis=1, keepdims=True))
        bm = jnp.where(lane == idx, -3.0e38, bm)
    w = jnp.concatenate(w_parts, axis=1)
    w = jax.nn.softmax(w, axis=-1)
    idx_ref[...] = jnp.concatenate(idx_parts, axis=1)
    w_ref[...] = w


def _agg_body(ll, top_k, tl, idx_ref, w_ref, v_ref, out_ref):
    b = pl.program_id(0)
    vcur = v_ref[0]
    shift0 = jax.lax.rem(ll - idx_ref[0, 0], ll)
    out_ref[0] = w_ref[b, 0] * pltpu.roll(vcur, shift0, axis=0)
    for i in range(1, top_k):
        shift = jax.lax.rem(ll - idx_ref[0, i], ll)
        out_ref[0] = out_ref[0] + w_ref[b, i] * pltpu.roll(vcur, shift, axis=0)


def kernel(queries, keys, values, attn_mask):
    B, L, H, E = queries.shape
    C = H * E
    T = 1024
    NB = L // T
    top_k = int(1 * math.log(L))

    q = queries.reshape(B, L, C)
    k = keys.reshape(B, L, C)
    v = values.reshape(B, L, C)

    low, up = pl.pallas_call(
        functools.partial(_corr_body, NB, T),
        grid=(B, NB, NB),
        in_specs=[
            pl.BlockSpec((1, 1, T, C), lambda b, j, a: (b, (a + j) % NB, 0, 0)),
            pl.BlockSpec((1, 1, T, C), lambda b, j, a: (b, a, 0, 0)),
        ],
        out_specs=[
            pl.BlockSpec((1, 1, T), lambda b, j, a: (b * NB + j, 0, 0)),
            pl.BlockSpec((1, 1, T), lambda b, j, a: (b * NB + j, 0, 0)),
        ],
        out_shape=[
            jax.ShapeDtypeStruct((B * NB, 1, T), jnp.float32),
            jax.ShapeDtypeStruct((B * NB, 1, T), jnp.float32),
        ],
        scratch_shapes=[pltpu.VMEM((T, T), jnp.float32)],
        compiler_params=pltpu.CompilerParams(
            dimension_semantics=("parallel", "parallel", "arbitrary"),
        ),
    )(q.reshape(B, NB, T, C), k.reshape(B, NB, T, C))

    low = low.reshape(B, L)
    up = up.reshape(B, L)

    idx, w = pl.pallas_call(
        functools.partial(_select_body, top_k, T, 1.0 / C),
        in_specs=[
            pl.BlockSpec((B, L), lambda: (0, 0)),
            pl.BlockSpec((B, L), lambda: (0, 0)),
        ],
        out_specs=[
            pl.BlockSpec((1, top_k), lambda: (0, 0)),
            pl.BlockSpec((B, top_k), lambda: (0, 0)),
        ],
        out_shape=[
            jax.ShapeDtypeStruct((1, top_k), jnp.int32),
            jax.ShapeDtypeStruct((B, top_k), jnp.float32),
        ],
    )(low, up)

    CB = 128
    NCB = C // CB
    TL = 512
    out = pl.pallas_call(
        functools.partial(_agg_body, L, top_k, TL),
        grid=(B, NCB),
        in_specs=[
            pl.BlockSpec(memory_space=pltpu.SMEM),
            pl.BlockSpec(memory_space=pltpu.SMEM),
            pl.BlockSpec((1, L, CB), lambda b, c: (b, 0, c)),
        ],
        out_specs=pl.BlockSpec((1, L, CB), lambda b, c: (b, 0, c)),
        out_shape=jax.ShapeDtypeStruct((B, L, C), jnp.float32),
        compiler_params=pltpu.CompilerParams(
            dimension_semantics=("parallel", "parallel"),
        ),
    )(idx, w, v)

    return out.reshape(B, L, H, E)
